# baseline (speedup 1.0000x reference)
import functools

import jax
import jax.numpy as jnp
from jax import lax
from jax.experimental import pallas as pl
from jax.experimental.pallas import tpu as pltpu
from jax.experimental.pallas import tpu_sc as plsc

B, S, C, P, N, D, M = 64, 336, 7, 96, 100000, 64, 16
ALPHA = 0.7
VROW = P * C
KB = 8192
NBLK = (N + KB - 1) // KB
CH_VREGS = 10
CH = CH_VREGS * 16
NEG = -3.0e38


def _sim_body(xf_ref, wq_ref, keys_ref, sim_ref, qn_ref):
    j = pl.program_id(0)

    @pl.when(j == 0)
    def _():
        q = jnp.dot(xf_ref[...], wq_ref[...], preferred_element_type=jnp.float32)
        qn = jnp.sqrt(jnp.sum(q * q, axis=1, keepdims=True))
        qn_ref[...] = q / (qn + 1e-6)

    kb = keys_ref[...]
    kn = jnp.sqrt(jnp.sum(kb * kb, axis=1, keepdims=True))
    kbn = kb * (1.0 / (kn + 1e-6))
    sim_ref[...] = lax.dot_general(
        qn_ref[...], kbn, (((1,), (1,)), ((), ())),
        preferred_element_type=jnp.float32)


def _compute_sim(xf, W_q, keys):
    return pl.pallas_call(
        _sim_body,
        grid=(NBLK,),
        in_specs=[
            pl.BlockSpec((B, S * C), lambda j: (0, 0)),
            pl.BlockSpec((S * C, D), lambda j: (0, 0)),
            pl.BlockSpec((KB, D), lambda j: (j, 0)),
        ],
        out_specs=pl.BlockSpec((B, KB), lambda j: (0, j)),
        out_shape=jax.ShapeDtypeStruct((B, N), jnp.float32),
        scratch_shapes=[pltpu.VMEM((B, D), jnp.float32)],
    )(xf, W_q, keys)


def _merge_sorted(tv, ti, sv, si):
    sel = tv >= sv
    z = jnp.where(sel, tv, sv)
    zi = jnp.where(sel, ti, si)
    tv2, ti2 = plsc.sort_key_val(z, zi, descending=True)
    return tv2, ti2


def _topk_body(sim_hbm, val_hbm, out_hbm, row_v, rows_v, acc_v, sem):
    wid = lax.axis_index("s") * 2 + lax.axis_index("c")
    for r in range(2):
        b = wid * 2 + r
        pltpu.sync_copy(sim_hbm.at[b], row_v)

        def scan_step(i, carry):
            tv, ti, thr = carry
            base = i * CH
            vs = [row_v[pl.ds(base + k * 16, 16)] for k in range(CH_VREGS)]
            mx = vs[0]
            for k in range(1, CH_VREGS):
                mx = jnp.maximum(mx, vs[k])

            def merge_path(c):
                tv, ti, thr = c
                for k in range(CH_VREGS):
                    v = vs[k]

                    def do(c2, _v=v, _k=k):
                        tv, ti, thr = c2
                        iv = lax.iota(jnp.int32, 16) + (base + _k * 16)
                        sv, si = plsc.sort_key_val(_v, iv, descending=False)
                        tv2, ti2 = _merge_sorted(tv, ti, sv, si)
                        return (tv2, ti2, jnp.min(tv2))

                    tv, ti, thr = lax.cond(jnp.max(v) > thr, do,
                                           lambda c2: c2, (tv, ti, thr))
                return (tv, ti, thr)

            return lax.cond(jnp.max(mx) > thr, merge_path,
                            lambda c: c, (tv, ti, thr))

        tv0 = jnp.full((16,), NEG, jnp.float32)
        ti0 = jnp.zeros((16,), jnp.int32)
        tv, ti, _ = lax.fori_loop(0, N // CH, scan_step, (tv0, ti0, NEG))

        s = tv / jnp.float32(1.0 - ALPHA + 1e-6)
        s = s - jnp.max(s)
        e = jnp.exp(s)
        w = e / jnp.sum(e)

        copies = [
            pltpu.async_copy(val_hbm.at[pl.ds(ti[m] * VROW, VROW)],
                             rows_v.at[m], sem)
            for m in range(M)
        ]
        for cp in copies:
            cp.wait()

        ws = [w[m] for m in range(M)]

        def wsum_step(jj, _):
            off = jj * 16
            acc = ws[0] * rows_v[0, pl.ds(off, 16)]
            for m in range(1, M):
                acc = acc + ws[m] * rows_v[m, pl.ds(off, 16)]
            acc_v[pl.ds(off, 16)] = acc
            return 0

        lax.fori_loop(0, VROW // 16, wsum_step, 0)
        pltpu.sync_copy(acc_v, out_hbm.at[b])


def _topk_aggregate(sim, values2d):
    mesh = plsc.VectorSubcoreMesh(core_axis_name="c", subcore_axis_name="s")
    kern = functools.partial(
        pl.kernel,
        mesh=mesh,
        out_type=jax.ShapeDtypeStruct((B, VROW), jnp.float32),
        scratch_types=[
            pltpu.VMEM((N,), jnp.float32),
            pltpu.VMEM((M, VROW), jnp.float32),
            pltpu.VMEM((VROW,), jnp.float32),
            pltpu.SemaphoreType.DMA,
        ],
        compiler_params=pltpu.CompilerParams(needs_layout_passes=False,
                                             use_tc_tiling_on_sc=False),
    )(_topk_body)
    return kern(sim, values2d)


def _fuse_body(xnT_ref, wx_ref, bx_ref, rt_ref, off_ref, wp1_ref, wp2_ref,
               bp_ref, out_ref):
    t1 = jnp.dot(xnT_ref[...], wx_ref[...], preferred_element_type=jnp.float32)
    t1 = t1 + bx_ref[...] + off_ref[...]
    t2 = rt_ref[...] + off_ref[...]
    out_ref[...] = (jnp.dot(t1, wp1_ref[...], preferred_element_type=jnp.float32)
                    + jnp.dot(t2, wp2_ref[...], preferred_element_type=jnp.float32)
                    + bp_ref[...])


def _fuse(xnT, W_x, b_x, rt448, off, Wp1, Wp2, b_pred):
    return pl.pallas_call(
        _fuse_body,
        out_shape=jax.ShapeDtypeStruct((B * C, P), jnp.float32),
    )(xnT, W_x, b_x.reshape(1, P), rt448, off, Wp1, Wp2, b_pred.reshape(1, P))


def kernel(x, keys, values, W_q, W_x, b_x, W_pred, b_pred):
    x_offset = x[:, -1:, :]
    x_norm = x - x_offset
    xf = x_norm.reshape(B, S * C)

    sim = _compute_sim(xf, W_q, keys)
    rt = _topk_aggregate(sim, values.reshape(N * VROW))

    rt448 = rt.reshape(B, P, C).transpose(0, 2, 1).reshape(B * C, P)
    xnT = x_norm.transpose(0, 2, 1).reshape(B * C, S)
    off = x_offset.reshape(B * C, 1)
    out448 = _fuse(xnT, W_x, b_x, rt448, off, W_pred[:P], W_pred[P:], b_pred)
    return out448.reshape(B, C, P).transpose(0, 2, 1)


# speedup vs baseline: 20.5439x; 20.5439x over previous
import functools

import jax
import jax.numpy as jnp
from jax import lax
from jax.experimental import pallas as pl
from jax.experimental.pallas import tpu as pltpu
from jax.experimental.pallas import tpu_sc as plsc

B, S, C, P, N, D, M = 64, 336, 7, 96, 100000, 64, 16
ALPHA = 0.7
VROW = P * C
NPAD = 102400
KBA = 10240
NBLKA = NPAD // KBA
TROWS = NPAD // 128
KBD = 4096
NBLKD = NPAD // KBD
NEG = -3.0e38


def _sim_body(xf_ref, wq_ref, keys_ref, sim_ref, qn_ref):
    j = pl.program_id(0)

    @pl.when(j == 0)
    def _():
        q = jnp.dot(xf_ref[...], wq_ref[...], preferred_element_type=jnp.float32)
        qn = jnp.sqrt(jnp.sum(q * q, axis=1, keepdims=True))
        qn_ref[...] = q / (qn + 1e-6)

    kbt = keys_ref[...]
    kn = jnp.sqrt(jnp.sum(kbt * kbt, axis=0, keepdims=True))
    kbn = kbt * (1.0 / (kn + 1e-6))
    sim = lax.dot_general(qn_ref[...], kbn, (((1,), (0,)), ((), ())),
                          preferred_element_type=jnp.float32)
    col = j * KBA + lax.broadcasted_iota(jnp.int32, (B, KBA), 1)
    sim = jnp.where(col < N, sim, NEG)
    sim_ref[...] = sim.reshape(B, KBA // 128, 128)


def _compute_sim(xf, W_q, keys):
    return pl.pallas_call(
        _sim_body,
        grid=(NBLKA,),
        in_specs=[
            pl.BlockSpec((B, S * C), lambda j: (0, 0)),
            pl.BlockSpec((S * C, D), lambda j: (0, 0)),
            pl.BlockSpec((D, KBA), lambda j: (0, j)),
        ],
        out_specs=pl.BlockSpec((B, KBA // 128, 128), lambda j: (0, j, 0)),
        out_shape=jax.ShapeDtypeStruct((B, TROWS, 128), jnp.float32),
        scratch_shapes=[pltpu.VMEM((B, D), jnp.float32)],
    )(xf, W_q, keys)


def _topk_body(sim_hbm, w_hbm, row_v):
    wid = lax.axis_index("s") * 2 + lax.axis_index("c")
    for r in range(2):
        b = wid * 2 + r
        pltpu.sync_copy(sim_hbm.at[b], row_v)

        def scan_step(t, carry):
            tv, ti, thr_v = carry
            vs = [row_v[t, pl.ds(k * 16, 16)] for k in range(8)]
            mx = vs[0]
            for k in range(1, 8):
                mx = jnp.maximum(mx, vs[k])
            nhit = plsc.all_reduce_population_count(mx > thr_v)

            def merge_path(c):
                tv, ti, thr_v = c
                for k in range(8):
                    v = vs[k]
                    nh = plsc.all_reduce_population_count(v > thr_v)

                    def do(c2, _v=v, _k=k):
                        tv, ti, _ = c2
                        iv = lax.iota(jnp.int32, 16) + (t * 128 + _k * 16)
                        sv, si = plsc.sort_key_val(_v, iv, descending=False)
                        sel = tv >= sv
                        z = jnp.where(sel, tv, sv)
                        zi = jnp.where(sel, ti, si)
                        tv2, ti2 = plsc.sort_key_val(z, zi, descending=True)
                        return (tv2, ti2, jnp.broadcast_to(tv2[15], (16,)))

                    tv, ti, thr_v = lax.cond(nh[0] > 0, do,
                                             lambda c2: c2, (tv, ti, thr_v))
                return (tv, ti, thr_v)

            return lax.cond(nhit[0] > 0, merge_path, lambda c: c,
                            (tv, ti, thr_v))

        tv0 = jnp.full((16,), NEG, jnp.float32)
        ti0 = jnp.zeros((16,), jnp.int32)
        thr0 = jnp.full((16,), NEG, jnp.float32)
        tv, ti, _ = lax.fori_loop(0, TROWS, scan_step, (tv0, ti0, thr0))

        s = tv / jnp.float32(1.0 - ALPHA + 1e-6)
        s = s - jnp.broadcast_to(s[0], (16,))
        e = jnp.exp(s)
        tot = plsc.cumsum(e)[15]
        w = e / jnp.broadcast_to(tot, (16,))

        zv = jnp.zeros((16,), jnp.float32)

        def zero_step(t, _):
            for k in range(8):
                row_v[t, pl.ds(k * 16, 16)] = zv
            return 0

        lax.fori_loop(0, TROWS, zero_step, 0)

        lane = lax.iota(jnp.int32, 16)
        for m in range(M):
            i_m = ti[m]
            t_m = i_m // 128
            l_m = i_m % 128
            base = (l_m // 16) * 16
            cur = row_v[t_m, pl.ds(base, 16)]
            row_v[t_m, pl.ds(base, 16)] = jnp.where(
                lane == (l_m - base), jnp.broadcast_to(w[m], (16,)), cur)

        pltpu.sync_copy(row_v, w_hbm.at[b])


def _topk_weights(sim3):
    mesh = plsc.VectorSubcoreMesh(core_axis_name="c", subcore_axis_name="s")
    kern = functools.partial(
        pl.kernel,
        mesh=mesh,
        out_type=jax.ShapeDtypeStruct((B, TROWS, 128), jnp.float32),
        scratch_types=[
            pltpu.VMEM((TROWS, 128), jnp.float32),
        ],
        compiler_params=pltpu.CompilerParams(needs_layout_passes=False),
    )(_topk_body)
    return kern(sim3)


def _agg_body(vt_ref, w_ref, out_ref, acc_ref):
    j = pl.program_id(0)

    @pl.when(j == 0)
    def _():
        acc_ref[...] = jnp.zeros_like(acc_ref)

    wblk = w_ref[...].reshape(B, KBD)

    @pl.when(j < NBLKD - 1)
    def _():
        acc_ref[...] += lax.dot_general(vt_ref[...], wblk,
                                        (((1,), (1,)), ((), ())),
                                        preferred_element_type=jnp.float32)

    @pl.when(j == NBLKD - 1)
    def _():
        col = j * KBD + lax.broadcasted_iota(jnp.int32, (VROW, KBD), 1)
        vblk = jnp.where(col < N, vt_ref[...], 0.0)
        acc = acc_ref[...] + lax.dot_general(vblk, wblk,
                                             (((1,), (1,)), ((), ())),
                                             preferred_element_type=jnp.float32)
        out_ref[...] = acc


def _aggregate(vt2, w3):
    return pl.pallas_call(
        _agg_body,
        grid=(NBLKD,),
        in_specs=[
            pl.BlockSpec((VROW, KBD), lambda j: (0, j)),
            pl.BlockSpec((B, KBD // 128, 128), lambda j: (0, j, 0)),
        ],
        out_specs=pl.BlockSpec((VROW, B), lambda j: (0, 0)),
        out_shape=jax.ShapeDtypeStruct((VROW, B), jnp.float32),
        scratch_shapes=[pltpu.VMEM((VROW, B), jnp.float32)],
    )(vt2, w3)


def _fuse_body(xnT_ref, wx_ref, bx_ref, rt_ref, off_ref, wp1_ref, wp2_ref,
               bp_ref, out_ref):
    t1 = jnp.dot(xnT_ref[...], wx_ref[...], preferred_element_type=jnp.float32)
    t1 = t1 + bx_ref[...] + off_ref[...]
    t2 = rt_ref[...] + off_ref[...]
    out_ref[...] = (jnp.dot(t1, wp1_ref[...], preferred_element_type=jnp.float32)
                    + jnp.dot(t2, wp2_ref[...], preferred_element_type=jnp.float32)
                    + bp_ref[...])


def _fuse(xnT, W_x, b_x, rt448, off, Wp1, Wp2, b_pred):
    return pl.pallas_call(
        _fuse_body,
        out_shape=jax.ShapeDtypeStruct((B * C, P), jnp.float32),
    )(xnT, W_x, b_x.reshape(1, P), rt448, off, Wp1, Wp2, b_pred.reshape(1, P))


def kernel(x, keys, values, W_q, W_x, b_x, W_pred, b_pred):
    x_offset = x[:, -1:, :]
    x_norm = x - x_offset
    xf = x_norm.reshape(B, S * C)

    sim3 = _compute_sim(xf, W_q, keys.T)
    w3 = _topk_weights(sim3)

    vt2 = values.transpose(2, 1, 0).reshape(C * P, N)

    rt_t = _aggregate(vt2, w3)

    rt448 = rt_t.reshape(C, P, B).transpose(2, 0, 1).reshape(B * C, P)
    xnT = x_norm.transpose(0, 2, 1).reshape(B * C, S)
    off = x_offset.reshape(B * C, 1)
    out448 = _fuse(xnT, W_x, b_x, rt448, off, W_pred[:P], W_pred[P:], b_pred)
    return out448.reshape(B, C, P).transpose(0, 2, 1)
